# Initial kernel scaffold; baseline (speedup 1.0000x reference)
import jax
import jax.numpy as jnp
from jax.experimental import pallas as pl


def kernel(h, edge_index, he, params):
    raise NotImplementedError("write your pallas kernel here")


# speedup vs baseline: 3.3206x; 3.3206x over previous
import functools

import jax
import jax.numpy as jnp
from jax import lax
from jax.experimental import pallas as pl
from jax.experimental.pallas import tpu as pltpu
from jax.experimental.pallas import tpu_sc as plsc

_N, _E, _D, _DE = 10000, 320000, 128, 16
_LAYERS, _DELTA, _HID = 4, 2.0, 512
_F32 = jnp.float32

_NC, _NS = 2, 16
_NW = _NC * _NS
_EPW = _E // _NS
_CH = 80
_NCHUNK = _EPW // _CH
_EP = 320512
_EPS = _EP // _NS
_CHS = 64
_NCHS = _EPS // _CHS
_HER = _EP * _DE // 128
_HRPC = _CHS * _DE // 128
_HEPT = _EPS * _DE // 128
_NP = 10112
_HALF = _NP // 2
_ACC = 5120
_ART = _ACC // _NS
_ZR = 160
_ORT = 320
_OLAST = _HALF - (_NS - 1) * _ORT


def _mesh():
    return plsc.VectorSubcoreMesh(core_axis_name="c", subcore_axis_name="s",
                                  num_cores=_NC, num_subcores=_NS)


def _fill(buf, value):
    rows, cols = buf.shape

    @pl.loop(0, rows)
    def _(i):
        @pl.loop(0, cols // 16)
        def _(j):
            buf[i, pl.ds(j * 16, 16)] = jnp.full((16,), value, _F32)


def _remap_dst(dst_seg, sidx_v, t, lo, ch):
    @pl.loop(0, ch // 16)
    def _(j):
        v = dst_seg[pl.ds(t * ch + j * 16, 16)] - lo
        ok = (v >= 0) & (v < _HALF)
        sidx_v[pl.ds(j * 16, 16)] = jnp.where(ok, v, _HALF)


def _zero_acc(acc, zer_v, s):
    _fill(zer_v, 0.0)

    @pl.loop(0, _ART // _ZR)
    def _(t):
        pltpu.sync_copy(zer_v, acc.at[pl.ds(s * _ART + t * _ZR, _ZR)])


def _copy_out(acc, out_hbm, s, lo):
    @pl.when(s < _NS - 1)
    def _():
        pltpu.sync_copy(acc.at[pl.ds(s * _ORT, _ORT)],
                        out_hbm.at[pl.ds(lo + s * _ORT, _ORT)])

    @pl.when(s == _NS - 1)
    def _():
        base = (_NS - 1) * _ORT
        pltpu.sync_copy(acc.at[pl.ds(base, _OLAST)],
                        out_hbm.at[pl.ds(lo + base, _OLAST)])


def _sc_edge_stats(he128, dst1):
    @functools.partial(
        pl.kernel,
        mesh=_mesh(),
        out_type=jax.ShapeDtypeStruct((_NP, _D), _F32),
        scratch_types=[
            pltpu.VMEM((_EPS,), jnp.int32),
            pltpu.VMEM((_CHS,), jnp.int32),
            pltpu.VMEM((_HRPC, _D), _F32),
            pltpu.VMEM((_CHS, _D), _F32),
            pltpu.VMEM((_ZR, _D), _F32),
            pltpu.VMEM_SHARED((_ACC, _D), _F32),
        ],
    )
    def k(he_hbm, dst_hbm, out_hbm, dst_seg, sidx_v, he_v, row_v, zer_v,
          acc):
        c = lax.axis_index("c")
        s = lax.axis_index("s")
        lo = c * _HALF
        _zero_acc(acc, zer_v, s)
        _fill(row_v, 0.0)
        one0 = jnp.where(lax.iota(jnp.int32, 16) == 0, 1.0, 0.0)

        @pl.loop(0, _CHS)
        def _(e):
            row_v[e, pl.ds(_DE, 16)] = one0

        pltpu.sync_copy(dst_hbm.at[pl.ds(s * _EPS, _EPS)], dst_seg)
        plsc.subcore_barrier()

        @pl.loop(0, _NCHS)
        def _(t):
            base = s * _HEPT + t * _HRPC
            pltpu.sync_copy(he_hbm.at[pl.ds(base, _HRPC)], he_v)
            for e in range(_CHS):
                row_v[e, pl.ds(0, _DE)] = he_v[e // 8,
                                               pl.ds((e % 8) * _DE, _DE)]
            _remap_dst(dst_seg, sidx_v, t, lo, _CHS)
            pltpu.sync_copy(row_v, acc.at[sidx_v], add=True)

        plsc.subcore_barrier()
        _copy_out(acc, out_hbm, s, lo)

    return k(he128, dst1)


def _sc_segsum(h, src1, dst1):
    @functools.partial(
        pl.kernel,
        mesh=_mesh(),
        out_type=jax.ShapeDtypeStruct((_NP, _D), _F32),
        scratch_types=[
            pltpu.VMEM((_EPW,), jnp.int32),
            pltpu.VMEM((_EPW,), jnp.int32),
            pltpu.VMEM((_CH,), jnp.int32),
            pltpu.VMEM((_CH, _D), _F32),
            pltpu.VMEM((_ZR, _D), _F32),
            pltpu.VMEM_SHARED((_ACC, _D), _F32),
            pltpu.SemaphoreType.DMA,
        ],
    )
    def k(h_hbm, src_hbm, dst_hbm, out_hbm, src_seg, dst_seg, sidx_v,
          rows_v, zer_v, acc, sem):
        c = lax.axis_index("c")
        s = lax.axis_index("s")
        lo = c * _HALF
        _zero_acc(acc, zer_v, s)
        pltpu.sync_copy(src_hbm.at[pl.ds(s * _EPW, _EPW)], src_seg)
        pltpu.sync_copy(dst_hbm.at[pl.ds(s * _EPW, _EPW)], dst_seg)
        plsc.subcore_barrier()

        @pl.loop(0, _NCHUNK)
        def _(t):
            gath = pltpu.async_copy(
                h_hbm.at[src_seg.at[pl.ds(t * _CH, _CH)]], rows_v, sem)
            _remap_dst(dst_seg, sidx_v, t, lo, _CH)
            gath.wait()
            pltpu.sync_copy(rows_v, acc.at[sidx_v], add=True)

        plsc.subcore_barrier()
        _copy_out(acc, out_hbm, s, lo)

    return k(h, src1, dst1)


_BN = 1000
_NB = _N // _BN


def _ln(x, g, b):
    m = jnp.mean(x, axis=-1, keepdims=True)
    v = jnp.mean((x - m) * (x - m), axis=-1, keepdims=True)
    return (x - m) / jnp.sqrt(v + 1e-5) * g + b


def _layer_body(h_r, sp_r, st_r, ws_r, we_r, bm_r, g1_r, b1_r,
                g2_r, b2_r, w1_r, c1_r, w2_r, c2_r, out_r):
    seg = sp_r[...]
    hes = st_r[:, 0:_DE]
    deg = st_r[:, _DE:_DE + 1]
    extra = jnp.dot(hes, we_r[...], preferred_element_type=_F32)
    extra = extra + deg * bm_r[...]
    agg = jnp.dot(seg, ws_r[...], preferred_element_type=_F32) + extra
    agg = agg / jnp.maximum(deg, _DELTA)
    h1 = h_r[...] + _ln(agg, g1_r[...], b1_r[...])
    y = _ln(h1, g2_r[...], b2_r[...])
    z = jax.nn.gelu(jnp.dot(y, w1_r[...], preferred_element_type=_F32)
                    + c1_r[...])
    out_r[...] = h1 + jnp.dot(z, w2_r[...], preferred_element_type=_F32) \
        + c2_r[...]


def _tc_layer(h, sp, stats, p):
    full = lambda shape: pl.BlockSpec(shape, lambda i: tuple(0 for _ in shape))
    call = pl.pallas_call(
        _layer_body,
        grid=(_NB,),
        in_specs=[
            pl.BlockSpec((_BN, _D), lambda i: (i, 0)),
            pl.BlockSpec((_BN, _D), lambda i: (i, 0)),
            pl.BlockSpec((_BN, _D), lambda i: (i, 0)),
            full((_D, _D)),
            full((_DE, _D)),
            full((1, _D)),
            full((1, _D)),
            full((1, _D)),
            full((1, _D)),
            full((1, _D)),
            full((_D, _HID)),
            full((1, _HID)),
            full((_HID, _D)),
            full((1, _D)),
        ],
        out_specs=pl.BlockSpec((_BN, _D), lambda i: (i, 0)),
        out_shape=jax.ShapeDtypeStruct((_N, _D), _F32),
    )
    row = lambda v: v.reshape(1, -1)
    return call(h, sp, stats, p["Ws"], p["We"], row(p["bm"]),
                row(p["ln1_g"]), row(p["ln1_b"]), row(p["ln2_g"]),
                row(p["ln2_b"]), p["W1"], row(p["b1"]), p["W2"],
                row(p["b2"]))


def _head_body(h_r, png_r, pnb_r, lw_r, lb_r, lng_r, lnb_r, local_r, g_r,
               acc_r):
    i = pl.program_id(0)
    local = _ln(h_r[...], png_r[...], pnb_r[...])
    local_r[...] = local

    @pl.when(i == 0)
    def _():
        acc_r[...] = jnp.zeros_like(acc_r)

    acc_r[...] += jnp.sum(local, axis=0, keepdims=True)

    @pl.when(i == _NB - 1)
    def _():
        pooled = acc_r[...] * (1.0 / _N)
        gg = jnp.dot(pooled, lw_r[...], preferred_element_type=_F32) \
            + lb_r[...]
        g_r[...] = _ln(gg, lng_r[...], lnb_r[...])


def _tc_head(h, params):
    full = lambda shape: pl.BlockSpec(shape, lambda i: tuple(0 for _ in shape))
    call = pl.pallas_call(
        _head_body,
        grid=(_NB,),
        in_specs=[
            pl.BlockSpec((_BN, _D), lambda i: (i, 0)),
            full((1, _D)),
            full((1, _D)),
            full((_D, _D)),
            full((1, _D)),
            full((1, _D)),
            full((1, _D)),
        ],
        out_specs=[
            pl.BlockSpec((_BN, _D), lambda i: (i, 0)),
            pl.BlockSpec((1, _D), lambda i: (0, 0)),
        ],
        out_shape=[
            jax.ShapeDtypeStruct((_N, _D), _F32),
            jax.ShapeDtypeStruct((1, _D), _F32),
        ],
        scratch_shapes=[pltpu.VMEM((1, _D), _F32)],
    )
    row = lambda v: v.reshape(1, -1)
    return call(h, row(params["pn_g"]), row(params["pn_b"]),
                params["lin_W"], row(params["lin_b"]),
                row(params["lin_ln_g"]), row(params["lin_ln_b"]))


def kernel(h, edge_index, he, params):
    src1 = edge_index[0]
    dst1 = edge_index[1]
    pad = _EP - _E
    he128 = jnp.pad(he, ((0, pad), (0, 0))).reshape(_HER, 128)
    dst_p = jnp.pad(dst1, (0, pad), constant_values=_N)
    stats = _sc_edge_stats(he128, dst_p)
    for i in range(_LAYERS):
        p = params["layer0"] if i == 0 else params["layer1"]
        sp = _sc_segsum(h, src1, dst1)
        h = _tc_layer(h, sp, stats, p)
    return _tc_head(h, params)


# speedup vs baseline: 4.2122x; 1.2685x over previous
import functools

import jax
import jax.numpy as jnp
from jax import lax
from jax.experimental import pallas as pl
from jax.experimental.pallas import tpu as pltpu
from jax.experimental.pallas import tpu_sc as plsc

_N, _E, _D, _DE = 10000, 320000, 128, 16
_LAYERS, _DELTA, _HID = 4, 2.0, 512
_F32 = jnp.float32

_NC, _NS = 2, 16
_NW = _NC * _NS
_EPW = _E // _NS
_CH = 80
_NCHUNK = _EPW // _CH
_EP = 320512
_EPS = _EP // _NS
_CHS = 64
_NCHS = _EPS // _CHS
_HER = _EP * _DE // 128
_HRPC = _CHS * _DE // 128
_HEPT = _EPS * _DE // 128
_NP = 10112
_HALF = _NP // 2
_ACC = 5120
_ART = _ACC // _NS
_ZR = 80
_NBUF = 2
_ORT = 320
_OLAST = _HALF - (_NS - 1) * _ORT


def _mesh():
    return plsc.VectorSubcoreMesh(core_axis_name="c", subcore_axis_name="s",
                                  num_cores=_NC, num_subcores=_NS)


def _fill(buf, value):
    rows, cols = buf.shape

    @pl.loop(0, rows)
    def _(i):
        @pl.loop(0, cols // 16)
        def _(j):
            buf[i, pl.ds(j * 16, 16)] = jnp.full((16,), value, _F32)


def _remap_dst(dst_seg, sidx_v, t, lo, ch):
    @pl.loop(0, ch // 16)
    def _(j):
        v = dst_seg[pl.ds(t * ch + j * 16, 16)] - lo
        ok = (v >= 0) & (v < _HALF)
        sidx_v[pl.ds(j * 16, 16)] = jnp.where(ok, v, _HALF)


def _zero_acc(acc, zer_v, s):
    _fill(zer_v, 0.0)

    @pl.loop(0, _ART // _ZR)
    def _(t):
        pltpu.sync_copy(zer_v, acc.at[pl.ds(s * _ART + t * _ZR, _ZR)])


def _copy_out(acc, out_hbm, s, lo):
    @pl.when(s < _NS - 1)
    def _():
        pltpu.sync_copy(acc.at[pl.ds(s * _ORT, _ORT)],
                        out_hbm.at[pl.ds(lo + s * _ORT, _ORT)])

    @pl.when(s == _NS - 1)
    def _():
        base = (_NS - 1) * _ORT
        pltpu.sync_copy(acc.at[pl.ds(base, _OLAST)],
                        out_hbm.at[pl.ds(lo + base, _OLAST)])


def _sc_edge_stats(he128, dst1):
    @functools.partial(
        pl.kernel,
        mesh=_mesh(),
        out_type=jax.ShapeDtypeStruct((_NP, _D), _F32),
        scratch_types=[
            pltpu.VMEM((_EPS,), jnp.int32),
            pltpu.VMEM((_CHS,), jnp.int32),
            pltpu.VMEM((_HRPC, _D), _F32),
            pltpu.VMEM((_CHS, _D), _F32),
            pltpu.VMEM((_ZR, _D), _F32),
            pltpu.VMEM_SHARED((_ACC, _D), _F32),
        ],
    )
    def k(he_hbm, dst_hbm, out_hbm, dst_seg, sidx_v, he_v, row_v, zer_v,
          acc):
        c = lax.axis_index("c")
        s = lax.axis_index("s")
        lo = c * _HALF
        _zero_acc(acc, zer_v, s)
        _fill(row_v, 0.0)
        one0 = jnp.where(lax.iota(jnp.int32, 16) == 0, 1.0, 0.0)

        @pl.loop(0, _CHS)
        def _(e):
            row_v[e, pl.ds(_DE, 16)] = one0

        pltpu.sync_copy(dst_hbm.at[pl.ds(s * _EPS, _EPS)], dst_seg)
        plsc.subcore_barrier()

        @pl.loop(0, _NCHS)
        def _(t):
            base = s * _HEPT + t * _HRPC
            pltpu.sync_copy(he_hbm.at[pl.ds(base, _HRPC)], he_v)
            for e in range(_CHS):
                row_v[e, pl.ds(0, _DE)] = he_v[e // 8,
                                               pl.ds((e % 8) * _DE, _DE)]
            _remap_dst(dst_seg, sidx_v, t, lo, _CHS)
            pltpu.sync_copy(row_v, acc.at[sidx_v], add=True)

        plsc.subcore_barrier()
        _copy_out(acc, out_hbm, s, lo)

    return k(he128, dst1)


def _sc_segsum(h, src1, dst1):
    @functools.partial(
        pl.kernel,
        mesh=_mesh(),
        out_type=jax.ShapeDtypeStruct((_NP, _D), _F32),
        scratch_types=[
            pltpu.VMEM((_EPW,), jnp.int32),
            pltpu.VMEM((_EPW,), jnp.int32),
            [pltpu.VMEM((_CH,), jnp.int32) for _ in range(_NBUF)],
            [pltpu.VMEM((_CH, _D), _F32) for _ in range(_NBUF)],
            pltpu.VMEM((_ZR, _D), _F32),
            pltpu.VMEM_SHARED((_ACC, _D), _F32),
            pltpu.SemaphoreType.DMA,
            pltpu.SemaphoreType.DMA,
        ],
    )
    def k(h_hbm, src_hbm, dst_hbm, out_hbm, src_seg, dst_seg, sidxs,
          bufs, zer_v, acc, gsem, ssem):
        c = lax.axis_index("c")
        s = lax.axis_index("s")
        lo = c * _HALF
        _zero_acc(acc, zer_v, s)
        pltpu.sync_copy(src_hbm.at[pl.ds(s * _EPW, _EPW)], src_seg)
        pltpu.sync_copy(dst_hbm.at[pl.ds(s * _EPW, _EPW)], dst_seg)
        plsc.subcore_barrier()

        @pl.loop(0, _NCHUNK, step=_NBUF)
        def _(t0):
            gath = []
            for b in range(_NBUF):
                gath.append(pltpu.async_copy(
                    h_hbm.at[src_seg.at[pl.ds((t0 + b) * _CH, _CH)]],
                    bufs[b], gsem))
            scat = []
            for b in range(_NBUF):
                _remap_dst(dst_seg, sidxs[b], t0 + b, lo, _CH)
                gath[b].wait()
                scat.append(pltpu.async_copy(bufs[b], acc.at[sidxs[b]],
                                             ssem, add=True))
            for b in range(_NBUF):
                scat[b].wait()

        plsc.subcore_barrier()
        _copy_out(acc, out_hbm, s, lo)

    return k(h, src1, dst1)


_BN = 1000
_NB = _N // _BN


def _ln(x, g, b):
    m = jnp.mean(x, axis=-1, keepdims=True)
    v = jnp.mean((x - m) * (x - m), axis=-1, keepdims=True)
    return (x - m) / jnp.sqrt(v + 1e-5) * g + b


def _layer_body(h_r, sp_r, st_r, ws_r, we_r, bm_r, g1_r, b1_r,
                g2_r, b2_r, w1_r, c1_r, w2_r, c2_r, out_r):
    seg = sp_r[...]
    hes = st_r[:, 0:_DE]
    deg = st_r[:, _DE:_DE + 1]
    extra = jnp.dot(hes, we_r[...], preferred_element_type=_F32)
    extra = extra + deg * bm_r[...]
    agg = jnp.dot(seg, ws_r[...], preferred_element_type=_F32) + extra
    agg = agg / jnp.maximum(deg, _DELTA)
    h1 = h_r[...] + _ln(agg, g1_r[...], b1_r[...])
    y = _ln(h1, g2_r[...], b2_r[...])
    z = jax.nn.gelu(jnp.dot(y, w1_r[...], preferred_element_type=_F32)
                    + c1_r[...])
    out_r[...] = h1 + jnp.dot(z, w2_r[...], preferred_element_type=_F32) \
        + c2_r[...]


def _tc_layer(h, sp, stats, p):
    full = lambda shape: pl.BlockSpec(shape, lambda i: tuple(0 for _ in shape))
    call = pl.pallas_call(
        _layer_body,
        grid=(_NB,),
        in_specs=[
            pl.BlockSpec((_BN, _D), lambda i: (i, 0)),
            pl.BlockSpec((_BN, _D), lambda i: (i, 0)),
            pl.BlockSpec((_BN, _D), lambda i: (i, 0)),
            full((_D, _D)),
            full((_DE, _D)),
            full((1, _D)),
            full((1, _D)),
            full((1, _D)),
            full((1, _D)),
            full((1, _D)),
            full((_D, _HID)),
            full((1, _HID)),
            full((_HID, _D)),
            full((1, _D)),
        ],
        out_specs=pl.BlockSpec((_BN, _D), lambda i: (i, 0)),
        out_shape=jax.ShapeDtypeStruct((_N, _D), _F32),
    )
    row = lambda v: v.reshape(1, -1)
    return call(h, sp, stats, p["Ws"], p["We"], row(p["bm"]),
                row(p["ln1_g"]), row(p["ln1_b"]), row(p["ln2_g"]),
                row(p["ln2_b"]), p["W1"], row(p["b1"]), p["W2"],
                row(p["b2"]))


def _head_body(h_r, png_r, pnb_r, lw_r, lb_r, lng_r, lnb_r, local_r, g_r,
               acc_r):
    i = pl.program_id(0)
    local = _ln(h_r[...], png_r[...], pnb_r[...])
    local_r[...] = local

    @pl.when(i == 0)
    def _():
        acc_r[...] = jnp.zeros_like(acc_r)

    acc_r[...] += jnp.sum(local, axis=0, keepdims=True)

    @pl.when(i == _NB - 1)
    def _():
        pooled = acc_r[...] * (1.0 / _N)
        gg = jnp.dot(pooled, lw_r[...], preferred_element_type=_F32) \
            + lb_r[...]
        g_r[...] = _ln(gg, lng_r[...], lnb_r[...])


def _tc_head(h, params):
    full = lambda shape: pl.BlockSpec(shape, lambda i: tuple(0 for _ in shape))
    call = pl.pallas_call(
        _head_body,
        grid=(_NB,),
        in_specs=[
            pl.BlockSpec((_BN, _D), lambda i: (i, 0)),
            full((1, _D)),
            full((1, _D)),
            full((_D, _D)),
            full((1, _D)),
            full((1, _D)),
            full((1, _D)),
        ],
        out_specs=[
            pl.BlockSpec((_BN, _D), lambda i: (i, 0)),
            pl.BlockSpec((1, _D), lambda i: (0, 0)),
        ],
        out_shape=[
            jax.ShapeDtypeStruct((_N, _D), _F32),
            jax.ShapeDtypeStruct((1, _D), _F32),
        ],
        scratch_shapes=[pltpu.VMEM((1, _D), _F32)],
    )
    row = lambda v: v.reshape(1, -1)
    return call(h, row(params["pn_g"]), row(params["pn_b"]),
                params["lin_W"], row(params["lin_b"]),
                row(params["lin_ln_g"]), row(params["lin_ln_b"]))


def kernel(h, edge_index, he, params):
    src1 = edge_index[0]
    dst1 = edge_index[1]
    pad = _EP - _E
    he128 = jnp.pad(he, ((0, pad), (0, 0))).reshape(_HER, 128)
    dst_p = jnp.pad(dst1, (0, pad), constant_values=_N)
    stats = _sc_edge_stats(he128, dst_p)
    for i in range(_LAYERS):
        p = params["layer0"] if i == 0 else params["layer1"]
        sp = _sc_segsum(h, src1, dst1)
        h = _tc_layer(h, sp, stats, p)
    return _tc_head(h, params)


# speedup vs baseline: 4.4594x; 1.0587x over previous
import functools

import jax
import jax.numpy as jnp
from jax import lax
from jax.experimental import pallas as pl
from jax.experimental.pallas import tpu as pltpu
from jax.experimental.pallas import tpu_sc as plsc

_N, _E, _D, _DE = 10000, 320000, 128, 16
_LAYERS, _DELTA, _HID = 4, 2.0, 512
_F32 = jnp.float32

_NC, _NS = 2, 16
_NW = _NC * _NS
_EPW = _E // _NS
_CH = 80
_NCHUNK = _EPW // _CH
_EP = 321536
_EPS = _EP // _NS
_CHS = 64
_NCHS = _EPS // _CHS
_HER = _EP * _DE // 128
_HRPC = _CHS * _DE // 128
_HEPT = _EPS * _DE // 128
_NP = 10112
_HALF = _NP // 2
_ACC = 5120
_ART = _ACC // _NS
_ZR = 80
_NBUF = 2
_ORT = 320
_OLAST = _HALF - (_NS - 1) * _ORT


def _mesh():
    return plsc.VectorSubcoreMesh(core_axis_name="c", subcore_axis_name="s",
                                  num_cores=_NC, num_subcores=_NS)


def _fill(buf, value):
    rows, cols = buf.shape

    @pl.loop(0, rows)
    def _(i):
        @pl.loop(0, cols // 16)
        def _(j):
            buf[i, pl.ds(j * 16, 16)] = jnp.full((16,), value, _F32)


def _remap_dst(dst_seg, sidx_v, t, lo, ch):
    @pl.loop(0, ch // 16)
    def _(j):
        v = dst_seg[pl.ds(t * ch + j * 16, 16)] - lo
        ok = (v >= 0) & (v < _HALF)
        sidx_v[pl.ds(j * 16, 16)] = jnp.where(ok, v, _HALF)


def _zero_acc(acc, zer_v, s):
    _fill(zer_v, 0.0)

    @pl.loop(0, _ART // _ZR)
    def _(t):
        pltpu.sync_copy(zer_v, acc.at[pl.ds(s * _ART + t * _ZR, _ZR)])


def _copy_out(acc, out_hbm, s, lo):
    @pl.when(s < _NS - 1)
    def _():
        pltpu.sync_copy(acc.at[pl.ds(s * _ORT, _ORT)],
                        out_hbm.at[pl.ds(lo + s * _ORT, _ORT)])

    @pl.when(s == _NS - 1)
    def _():
        base = (_NS - 1) * _ORT
        pltpu.sync_copy(acc.at[pl.ds(base, _OLAST)],
                        out_hbm.at[pl.ds(lo + base, _OLAST)])


def _sc_edge_stats(he128, dst1):
    @functools.partial(
        pl.kernel,
        mesh=_mesh(),
        out_type=jax.ShapeDtypeStruct((_NP, _D), _F32),
        scratch_types=[
            pltpu.VMEM((_EPS,), jnp.int32),
            [pltpu.VMEM((_CHS,), jnp.int32) for _ in range(2)],
            [pltpu.VMEM((_HRPC, _D), _F32) for _ in range(2)],
            [pltpu.VMEM((_CHS, _D), _F32) for _ in range(2)],
            pltpu.VMEM((_ZR, _D), _F32),
            pltpu.VMEM_SHARED((_ACC, _D), _F32),
            pltpu.SemaphoreType.DMA,
            pltpu.SemaphoreType.DMA,
        ],
    )
    def k(he_hbm, dst_hbm, out_hbm, dst_seg, sidxs, he_bufs, row_bufs,
          zer_v, acc, hsem, ssem):
        c = lax.axis_index("c")
        s = lax.axis_index("s")
        lo = c * _HALF
        _zero_acc(acc, zer_v, s)
        one0 = jnp.where(lax.iota(jnp.int32, 16) == 0, 1.0, 0.0)
        for b in range(2):
            _fill(row_bufs[b], 0.0)

            @pl.loop(0, _CHS)
            def _(e):
                row_bufs[b][e, pl.ds(_DE, 16)] = one0

        pltpu.sync_copy(dst_hbm.at[pl.ds(s * _EPS, _EPS)], dst_seg)
        plsc.subcore_barrier()

        @pl.loop(0, _NCHS, step=2)
        def _(t0):
            lds = []
            for b in range(2):
                base = s * _HEPT + (t0 + b) * _HRPC
                lds.append(pltpu.async_copy(
                    he_hbm.at[pl.ds(base, _HRPC)], he_bufs[b], hsem))
            scat = []
            for b in range(2):
                lds[b].wait()
                for e in range(_CHS):
                    row_bufs[b][e, pl.ds(0, _DE)] = \
                        he_bufs[b][e // 8, pl.ds((e % 8) * _DE, _DE)]
                _remap_dst(dst_seg, sidxs[b], t0 + b, lo, _CHS)
                scat.append(pltpu.async_copy(
                    row_bufs[b], acc.at[sidxs[b]], ssem, add=True))
            for b in range(2):
                scat[b].wait()

        plsc.subcore_barrier()
        _copy_out(acc, out_hbm, s, lo)

    return k(he128, dst1)


def _sc_segsum(h, src1, dst1):
    @functools.partial(
        pl.kernel,
        mesh=_mesh(),
        out_type=jax.ShapeDtypeStruct((_NP, _D), _F32),
        scratch_types=[
            pltpu.VMEM((_EPW,), jnp.int32),
            pltpu.VMEM((_EPW,), jnp.int32),
            [pltpu.VMEM((_CH,), jnp.int32) for _ in range(_NBUF)],
            [pltpu.VMEM((_CH, _D), _F32) for _ in range(_NBUF)],
            pltpu.VMEM((_ZR, _D), _F32),
            pltpu.VMEM_SHARED((_ACC, _D), _F32),
            pltpu.SemaphoreType.DMA,
            pltpu.SemaphoreType.DMA,
        ],
    )
    def k(h_hbm, src_hbm, dst_hbm, out_hbm, src_seg, dst_seg, sidxs,
          bufs, zer_v, acc, gsem, ssem):
        c = lax.axis_index("c")
        s = lax.axis_index("s")
        lo = c * _HALF
        _zero_acc(acc, zer_v, s)
        pltpu.sync_copy(src_hbm.at[pl.ds(s * _EPW, _EPW)], src_seg)
        pltpu.sync_copy(dst_hbm.at[pl.ds(s * _EPW, _EPW)], dst_seg)
        plsc.subcore_barrier()

        @pl.loop(0, _NCHUNK, step=_NBUF)
        def _(t0):
            gath = []
            for b in range(_NBUF):
                gath.append(pltpu.async_copy(
                    h_hbm.at[src_seg.at[pl.ds((t0 + b) * _CH, _CH)]],
                    bufs[b], gsem))
            scat = []
            for b in range(_NBUF):
                _remap_dst(dst_seg, sidxs[b], t0 + b, lo, _CH)
                gath[b].wait()
                scat.append(pltpu.async_copy(bufs[b], acc.at[sidxs[b]],
                                             ssem, add=True))
            for b in range(_NBUF):
                scat[b].wait()

        plsc.subcore_barrier()
        _copy_out(acc, out_hbm, s, lo)

    return k(h, src1, dst1)


_BN = 1000
_NB = _N // _BN


def _ln(x, g, b):
    m = jnp.mean(x, axis=-1, keepdims=True)
    v = jnp.mean((x - m) * (x - m), axis=-1, keepdims=True)
    return (x - m) / jnp.sqrt(v + 1e-5) * g + b


def _layer_body(h_r, sp_r, st_r, ws_r, we_r, bm_r, g1_r, b1_r,
                g2_r, b2_r, w1_r, c1_r, w2_r, c2_r, out_r):
    seg = sp_r[...]
    hes = st_r[:, 0:_DE]
    deg = st_r[:, _DE:_DE + 1]
    extra = jnp.dot(hes, we_r[...], preferred_element_type=_F32)
    extra = extra + deg * bm_r[...]
    agg = jnp.dot(seg, ws_r[...], preferred_element_type=_F32) + extra
    agg = agg / jnp.maximum(deg, _DELTA)
    h1 = h_r[...] + _ln(agg, g1_r[...], b1_r[...])
    y = _ln(h1, g2_r[...], b2_r[...])
    z = jax.nn.gelu(jnp.dot(y, w1_r[...], preferred_element_type=_F32)
                    + c1_r[...])
    out_r[...] = h1 + jnp.dot(z, w2_r[...], preferred_element_type=_F32) \
        + c2_r[...]


def _tc_layer(h, sp, stats, p):
    full = lambda shape: pl.BlockSpec(shape, lambda i: tuple(0 for _ in shape))
    call = pl.pallas_call(
        _layer_body,
        grid=(_NB,),
        in_specs=[
            pl.BlockSpec((_BN, _D), lambda i: (i, 0)),
            pl.BlockSpec((_BN, _D), lambda i: (i, 0)),
            pl.BlockSpec((_BN, _D), lambda i: (i, 0)),
            full((_D, _D)),
            full((_DE, _D)),
            full((1, _D)),
            full((1, _D)),
            full((1, _D)),
            full((1, _D)),
            full((1, _D)),
            full((_D, _HID)),
            full((1, _HID)),
            full((_HID, _D)),
            full((1, _D)),
        ],
        out_specs=pl.BlockSpec((_BN, _D), lambda i: (i, 0)),
        out_shape=jax.ShapeDtypeStruct((_N, _D), _F32),
    )
    row = lambda v: v.reshape(1, -1)
    return call(h, sp, stats, p["Ws"], p["We"], row(p["bm"]),
                row(p["ln1_g"]), row(p["ln1_b"]), row(p["ln2_g"]),
                row(p["ln2_b"]), p["W1"], row(p["b1"]), p["W2"],
                row(p["b2"]))


def _head_body(h_r, png_r, pnb_r, lw_r, lb_r, lng_r, lnb_r, local_r, g_r,
               acc_r):
    i = pl.program_id(0)
    local = _ln(h_r[...], png_r[...], pnb_r[...])
    local_r[...] = local

    @pl.when(i == 0)
    def _():
        acc_r[...] = jnp.zeros_like(acc_r)

    acc_r[...] += jnp.sum(local, axis=0, keepdims=True)

    @pl.when(i == _NB - 1)
    def _():
        pooled = acc_r[...] * (1.0 / _N)
        gg = jnp.dot(pooled, lw_r[...], preferred_element_type=_F32) \
            + lb_r[...]
        g_r[...] = _ln(gg, lng_r[...], lnb_r[...])


def _tc_head(h, params):
    full = lambda shape: pl.BlockSpec(shape, lambda i: tuple(0 for _ in shape))
    call = pl.pallas_call(
        _head_body,
        grid=(_NB,),
        in_specs=[
            pl.BlockSpec((_BN, _D), lambda i: (i, 0)),
            full((1, _D)),
            full((1, _D)),
            full((_D, _D)),
            full((1, _D)),
            full((1, _D)),
            full((1, _D)),
        ],
        out_specs=[
            pl.BlockSpec((_BN, _D), lambda i: (i, 0)),
            pl.BlockSpec((1, _D), lambda i: (0, 0)),
        ],
        out_shape=[
            jax.ShapeDtypeStruct((_N, _D), _F32),
            jax.ShapeDtypeStruct((1, _D), _F32),
        ],
        scratch_shapes=[pltpu.VMEM((1, _D), _F32)],
    )
    row = lambda v: v.reshape(1, -1)
    return call(h, row(params["pn_g"]), row(params["pn_b"]),
                params["lin_W"], row(params["lin_b"]),
                row(params["lin_ln_g"]), row(params["lin_ln_b"]))


def kernel(h, edge_index, he, params):
    src1 = edge_index[0]
    dst1 = edge_index[1]
    pad = _EP - _E
    he128 = jnp.pad(he, ((0, pad), (0, 0))).reshape(_HER, 128)
    dst_p = jnp.pad(dst1, (0, pad), constant_values=_N)
    stats = _sc_edge_stats(he128, dst_p)
    for i in range(_LAYERS):
        p = params["layer0"] if i == 0 else params["layer1"]
        sp = _sc_segsum(h, src1, dst1)
        h = _tc_layer(h, sp, stats, p)
    return _tc_head(h, params)
